# Initial kernel scaffold; baseline (speedup 1.0000x reference)
import jax
import jax.numpy as jnp
from jax.experimental import pallas as pl


def kernel(H, weight, bias):
    raise NotImplementedError("write your pallas kernel here")


# speedup vs baseline: 1.3310x; 1.3310x over previous
import functools

import jax
import jax.numpy as jnp
from jax.experimental import pallas as pl
from jax.experimental.pallas import tpu as pltpu


def _pool_kernel(h_ref, w_ref, o_ref, *, feat, npack, npos):
    D, P, C = feat, npack, npos
    S = P * C

    t = jnp.tanh(h_ref[...])
    prod = t * w_ref[...].reshape(1, 1, 128)

    parts = [
        jnp.sum(prod[:, :, p * D:(p + 1) * D], axis=-1) for p in range(P)
    ]
    scores = jnp.concatenate(parts, axis=-1) if P > 1 else parts[0]

    m = jnp.max(scores, axis=-1, keepdims=True)
    e = jnp.exp(scores - m)
    den = jnp.sum(e, axis=-1, keepdims=True)
    alpha = e / den

    if P > 1:
        k = jax.lax.broadcasted_iota(jnp.int32, (S, S), 0)
        s = jax.lax.broadcasted_iota(jnp.int32, (S, S), 1)
        perm = (P * (k % C) + k // C == s).astype(jnp.float32)
        alpha = jax.lax.dot(alpha, perm,
                            precision=jax.lax.Precision.HIGHEST,
                            preferred_element_type=jnp.float32)
    o_ref[...] = alpha


def _pool_dense_kernel(h_ref, w_ref, o_ref):
    t = jnp.tanh(h_ref[...])
    prod = t * w_ref[...].reshape(1, 1, -1)
    scores = jnp.sum(prod, axis=-1)
    m = jnp.max(scores, axis=-1, keepdims=True)
    e = jnp.exp(scores - m)
    den = jnp.sum(e, axis=-1, keepdims=True)
    o_ref[...] = e / den


def kernel(H, weight, bias):
    B, S, D = H.shape
    del bias
    w32 = weight.reshape(1, D).astype(jnp.float32)

    packable = (D < 128) and (128 % D == 0) and (S % (128 // D) == 0)

    if packable:
        P = 128 // D
        C = S // P
        TB = min(B, 128)
        if B % TB:
            TB = min(B, 8)
        grid = (pl.cdiv(B, TB),)
        cparams = pltpu.CompilerParams(
            dimension_semantics=("parallel",),
            vmem_limit_bytes=64 << 20,
        )
        Hp = H.reshape(B, C, 128)
        w_tiled = jnp.tile(w32, (1, P))
        out = pl.pallas_call(
            functools.partial(_pool_kernel, feat=D, npack=P, npos=C),
            out_shape=jax.ShapeDtypeStruct((B, S), jnp.float32),
            grid=grid,
            in_specs=[
                pl.BlockSpec((TB, C, 128), lambda b: (b, 0, 0)),
                pl.BlockSpec((1, 128), lambda b: (0, 0)),
            ],
            out_specs=pl.BlockSpec((TB, S), lambda b: (b, 0)),
            compiler_params=cparams,
        )(Hp, w_tiled)
        return out.reshape(B, 1, S).astype(H.dtype)

    TB = min(B, 8)
    grid = (pl.cdiv(B, TB),)
    cparams = pltpu.CompilerParams(
        dimension_semantics=("parallel",),
        vmem_limit_bytes=64 << 20,
    )
    out = pl.pallas_call(
        _pool_dense_kernel,
        out_shape=jax.ShapeDtypeStruct((B, S), jnp.float32),
        grid=grid,
        in_specs=[
            pl.BlockSpec((TB, S, D), lambda b: (b, 0, 0)),
            pl.BlockSpec((1, D), lambda b: (0, 0)),
        ],
        out_specs=pl.BlockSpec((TB, S), lambda b: (b, 0)),
        compiler_params=cparams,
    )(H, w32)
    return out.reshape(B, 1, S).astype(H.dtype)
